# Initial kernel scaffold; baseline (speedup 1.0000x reference)
import jax
import jax.numpy as jnp
from jax.experimental import pallas as pl


def kernel(x, indices):
    raise NotImplementedError("write your pallas kernel here")


# speedup vs baseline: 1.6281x; 1.6281x over previous
import jax
import jax.numpy as jnp
from jax import lax
from jax.experimental import pallas as pl
from jax.experimental.pallas import tpu as pltpu
from jax.experimental.pallas import tpu_sc as plsc

N = 100000
C = 128
L = 4 * N - 6
FLAT = L * C
TOT = N * C
NCORE = 2
NSUB = 16
NW = NCORE * NSUB
PER_TILE = TOT // NW
B = 8000
NCHUNK = PER_TILE // B
NV = B // 16
NSLOT = 2


def _vsel(v, idxs):
    return jnp.take_along_axis(v, idxs, axis=0, mode="promise_in_bounds")


def _lane_val(v, i):
    iota16 = lax.iota(jnp.int32, 16)
    return jnp.sum(jnp.where(iota16 == i, v, jnp.float32(0.0)))


def _popcount(mask):
    r = plsc.all_reduce_population_count(mask)
    return jnp.max(r) if getattr(r, "ndim", 0) else r


def _body(skey_hbm, sval_hbm, y_hbm,
          k0, k1, x0, x1, fk_ref, fv_ref, s0, s1):
    kbufs = (k0, k1)
    xbufs = (x0, x1)
    sems = (s0, s1)
    cid = lax.axis_index("c")
    sid = lax.axis_index("s")
    wid = sid * NCORE + cid
    base = wid * PER_TILE
    iota16 = lax.iota(jnp.int32, 16)

    def peek_run_end(p_end):
        def live():
            pe = pl.multiple_of(p_end, 16)
            pltpu.sync_copy(skey_hbm.at[pl.ds(pe, 16)], fk_ref)
            pltpu.sync_copy(sval_hbm.at[pl.ds(pe, 16)], fv_ref)
            fk = fk_ref[...]
            fv = fv_ref[...]
            kb = jnp.min(fk)
            kbv = jnp.broadcast_to(kb, (16,))
            neq = _popcount(fk == kbv)
            cand0 = _lane_val(fv, neq - 1)
            q0 = p_end + 16
            cont0 = jnp.logical_and(neq == 16, q0 + 16 <= TOT)

            def cond(st):
                return st[2]

            def bodyw(st):
                q, cand, _ = st
                qa = pl.multiple_of(q, 16)
                pltpu.sync_copy(skey_hbm.at[pl.ds(qa, 16)], fk_ref)
                pltpu.sync_copy(sval_hbm.at[pl.ds(qa, 16)], fv_ref)
                fk2 = fk_ref[...]
                fv2 = fv_ref[...]
                n2 = _popcount(fk2 == kbv)
                cand2 = jnp.where(n2 > 0, _lane_val(fv2, n2 - 1), cand)
                cont2 = jnp.logical_and(n2 == 16, q + 32 <= TOT)
                return (q + 16, cand2, cont2)

            _, cand, _ = lax.while_loop(cond, bodyw, (q0, cand0, cont0))
            return kb, cand

        def dead():
            return jnp.int32(-1), jnp.float32(0.0)

        return lax.cond(p_end < TOT, live, dead)

    def process_chunk(slot, t):
        kbuf, xbuf = kbufs[slot], xbufs[slot]
        p0 = pl.multiple_of(base + t * B, 16)
        pltpu.sync_copy(skey_hbm.at[pl.ds(p0, B)], kbuf)
        pltpu.sync_copy(sval_hbm.at[pl.ds(p0, B)], xbuf)

        kb, cand = peek_run_end(p0 + B)
        nk0 = jnp.broadcast_to(kb, (16,))
        nv0 = jnp.broadcast_to(cand, (16,))

        def bwd(u, carry):
            nk, nv = carry
            off = (NV - 1 - u) * 16
            kv = kbuf[pl.ds(off, 16)]
            cur = xbuf[pl.ds(off, 16)]
            for d in (1, 2, 4, 8, 8):
                idxs = (iota16 + d) & 15
                use_next = iota16 >= (16 - d)
                kd = jnp.where(use_next, _vsel(nk, idxs), _vsel(kv, idxs))
                xd = jnp.where(use_next, _vsel(nv, idxs), _vsel(cur, idxs))
                cur = jnp.where(kv == kd, xd, cur)
            xbuf[pl.ds(off, 16)] = cur
            return (kv, cur)

        lax.fori_loop(0, NV, bwd, (nk0, nv0))
        pltpu.async_copy(xbuf, y_hbm.at[kbuf], sems[slot])

    def superchunk(s, carry):
        for slot in range(NSLOT):
            @pl.when(s > 0)
            def _():
                pltpu.make_async_copy(xbufs[slot], y_hbm.at[kbufs[slot]],
                                      sems[slot]).wait()
            process_chunk(slot, s * NSLOT + slot)
        return carry

    lax.fori_loop(0, NCHUNK // NSLOT, superchunk, 0)
    for slot in range(NSLOT):
        pltpu.make_async_copy(xbufs[slot], y_hbm.at[kbufs[slot]],
                              sems[slot]).wait()


_scatter = pl.kernel(
    _body,
    out_type=(),
    mesh=plsc.VectorSubcoreMesh(core_axis_name="c", subcore_axis_name="s"),
    compiler_params=pltpu.CompilerParams(needs_layout_passes=False),
    scratch_types=(
        [pltpu.VMEM((B,), jnp.int32) for _ in range(NSLOT)]
        + [pltpu.VMEM((B,), jnp.float32) for _ in range(NSLOT)]
        + [pltpu.VMEM((16,), jnp.int32), pltpu.VMEM((16,), jnp.float32)]
        + [pltpu.SemaphoreType.DMA for _ in range(NSLOT)]
    ),
)


@jax.jit
def kernel(x, indices):
    idx32 = indices.astype(jnp.int32)
    keys = (idx32 * C + jnp.arange(C, dtype=jnp.int32)[None, :]).reshape(-1)
    skey, sval = lax.sort_key_val(keys, x.reshape(-1), is_stable=False)
    yref = jax.new_ref(jnp.zeros((FLAT,), jnp.float32))
    _scatter(skey, sval, yref)
    return yref[...].reshape(L, C)


# speedup vs baseline: 1.7006x; 1.0445x over previous
import jax
import jax.numpy as jnp
from jax import lax
from jax.experimental import pallas as pl
from jax.experimental.pallas import tpu as pltpu
from jax.experimental.pallas import tpu_sc as plsc

N = 100000
C = 128
L = 4 * N - 6
FLAT = L * C
TOT = N * C
NCORE = 2
NSUB = 16
NW = NCORE * NSUB
PER_TILE = TOT // NW
B = 8000
NCHUNK = PER_TILE // B
NV = B // 16
NSLOT = 5


def _vsel(v, idxs):
    return jnp.take_along_axis(v, idxs, axis=0, mode="promise_in_bounds")


def _lane_val(v, i):
    iota16 = lax.iota(jnp.int32, 16)
    return jnp.sum(jnp.where(iota16 == i, v, jnp.float32(0.0)))


def _popcount(mask):
    r = plsc.all_reduce_population_count(mask)
    return jnp.max(r) if getattr(r, "ndim", 0) else r


def _body(skey_hbm, sval_hbm, y_hbm,
          k0, k1, k2, k3, k4, x0, x1, x2, x3, x4,
          fk_ref, fv_ref, s0, s1, s2, s3, s4):
    kbufs = (k0, k1, k2, k3, k4)
    xbufs = (x0, x1, x2, x3, x4)
    sems = (s0, s1, s2, s3, s4)
    cid = lax.axis_index("c")
    sid = lax.axis_index("s")
    wid = sid * NCORE + cid
    base = wid * PER_TILE
    iota16 = lax.iota(jnp.int32, 16)

    def peek_run_end(p_end):
        def live():
            pe = pl.multiple_of(p_end, 16)
            pltpu.sync_copy(skey_hbm.at[pl.ds(pe, 16)], fk_ref)
            pltpu.sync_copy(sval_hbm.at[pl.ds(pe, 16)], fv_ref)
            fk = fk_ref[...]
            fv = fv_ref[...]
            kb = jnp.min(fk)
            kbv = jnp.broadcast_to(kb, (16,))
            neq = _popcount(fk == kbv)
            cand0 = _lane_val(fv, neq - 1)
            q0 = p_end + 16
            cont0 = jnp.logical_and(neq == 16, q0 + 16 <= TOT)

            def cond(st):
                return st[2]

            def bodyw(st):
                q, cand, _ = st
                qa = pl.multiple_of(q, 16)
                pltpu.sync_copy(skey_hbm.at[pl.ds(qa, 16)], fk_ref)
                pltpu.sync_copy(sval_hbm.at[pl.ds(qa, 16)], fv_ref)
                fk2 = fk_ref[...]
                fv2 = fv_ref[...]
                n2 = _popcount(fk2 == kbv)
                cand2 = jnp.where(n2 > 0, _lane_val(fv2, n2 - 1), cand)
                cont2 = jnp.logical_and(n2 == 16, q + 32 <= TOT)
                return (q + 16, cand2, cont2)

            _, cand, _ = lax.while_loop(cond, bodyw, (q0, cand0, cont0))
            return kb, cand

        def dead():
            return jnp.int32(-1), jnp.float32(0.0)

        return lax.cond(p_end < TOT, live, dead)

    def process_chunk(slot, t):
        kbuf, xbuf = kbufs[slot], xbufs[slot]
        p0 = pl.multiple_of(base + t * B, 16)
        pltpu.sync_copy(skey_hbm.at[pl.ds(p0, B)], kbuf)
        pltpu.sync_copy(sval_hbm.at[pl.ds(p0, B)], xbuf)

        kb, cand = peek_run_end(p0 + B)
        nk0 = jnp.broadcast_to(kb, (16,))
        nv0 = jnp.broadcast_to(cand, (16,))

        def bwd(u, carry):
            nk, nv = carry
            off = (NV - 1 - u) * 16
            kv = kbuf[pl.ds(off, 16)]
            cur = xbuf[pl.ds(off, 16)]
            for d in (1, 2, 4, 8, 8):
                idxs = (iota16 + d) & 15
                use_next = iota16 >= (16 - d)
                kd = jnp.where(use_next, _vsel(nk, idxs), _vsel(kv, idxs))
                xd = jnp.where(use_next, _vsel(nv, idxs), _vsel(cur, idxs))
                cur = jnp.where(kv == kd, xd, cur)
            xbuf[pl.ds(off, 16)] = cur
            return (kv, cur)

        lax.fori_loop(0, NV, bwd, (nk0, nv0))
        pltpu.async_copy(xbuf, y_hbm.at[kbuf], sems[slot])

    def superchunk(s, carry):
        for slot in range(NSLOT):
            @pl.when(s > 0)
            def _():
                pltpu.make_async_copy(xbufs[slot], y_hbm.at[kbufs[slot]],
                                      sems[slot]).wait()
            process_chunk(slot, s * NSLOT + slot)
        return carry

    lax.fori_loop(0, NCHUNK // NSLOT, superchunk, 0)
    for slot in range(NSLOT):
        pltpu.make_async_copy(xbufs[slot], y_hbm.at[kbufs[slot]],
                              sems[slot]).wait()


_scatter = pl.kernel(
    _body,
    out_type=(),
    mesh=plsc.VectorSubcoreMesh(core_axis_name="c", subcore_axis_name="s"),
    compiler_params=pltpu.CompilerParams(needs_layout_passes=False),
    scratch_types=(
        [pltpu.VMEM((B,), jnp.int32) for _ in range(NSLOT)]
        + [pltpu.VMEM((B,), jnp.float32) for _ in range(NSLOT)]
        + [pltpu.VMEM((16,), jnp.int32), pltpu.VMEM((16,), jnp.float32)]
        + [pltpu.SemaphoreType.DMA for _ in range(NSLOT)]
    ),
)


@jax.jit
def kernel(x, indices):
    idx32 = indices.astype(jnp.int32)
    keys = (idx32 * C + jnp.arange(C, dtype=jnp.int32)[None, :]).reshape(-1)
    skey, sval = lax.sort_key_val(keys, x.reshape(-1), is_stable=False)
    yref = jax.new_ref(jnp.zeros((FLAT,), jnp.float32))
    _scatter(skey, sval, yref)
    return yref[...].reshape(L, C)


# speedup vs baseline: 4.0711x; 2.3939x over previous
import jax
import jax.numpy as jnp
from jax import lax
from jax.experimental import pallas as pl
from jax.experimental.pallas import tpu as pltpu
from jax.experimental.pallas import tpu_sc as plsc

N = 100000
C = 128
L = 4 * N - 6
FLAT = L * C
TOT = N * C
NCORE = 2
NSUB = 16
NW = NCORE * NSUB
WPT = 1599984
WLAST = FLAT - 31 * WPT
WSIZE = 48000
NWIN = 33
TAIL = WPT - NWIN * WSIZE
TAIL31 = WLAST - NWIN * WSIZE
ESIZE = 16384


def _vsel(v, idxs):
    return jnp.take_along_axis(v, idxs, axis=0, mode="promise_in_bounds")


def _popcount(mask):
    r = plsc.all_reduce_population_count(mask)
    return jnp.max(r) if getattr(r, "ndim", 0) else r


def _body(skey_hbm, sval_hbm, starts_hbm, y_hbm,
          sb0, sb1, kabuf, xabuf, svec, sem0, sem1):
    sbufs = (sb0, sb1)
    sems = (sem0, sem1)
    cid = lax.axis_index("c")
    sid = lax.axis_index("s")
    wid = sid * NCORE + cid
    iota16 = lax.iota(jnp.int32, 16)
    zeros16 = jnp.zeros((16,), jnp.float32)

    pltpu.sync_copy(starts_hbm.at[wid], svec)
    es = jnp.min(svec[...])
    g0 = es - lax.bitwise_and(es, 15)
    tile_w0 = wid * WPT

    def consume_window(g_init, w0s, w1s, sb, wsize):
        def refill_cond(st):
            return jnp.logical_not(st[1])

        def refill_body(st):
            g_, _ = st
            gb = jnp.minimum(g_, TOT - ESIZE)
            gba = pl.multiple_of(gb, 16)
            pltpu.sync_copy(skey_hbm.at[pl.ds(gba, ESIZE)], kabuf)
            pltpu.sync_copy(sval_hbm.at[pl.ds(gba, ESIZE)], xabuf)

            def cond(st2):
                return st2[2]

            def body(st2):
                g2, _, _ = st2
                off = g2 - gb
                kv = kabuf[pl.ds(off, 16)]
                cur = xabuf[pl.ds(off, 16)]
                for d in (1, 2, 4, 8):
                    idxs = jnp.minimum(iota16 + d, 15)
                    kd = _vsel(kv, idxs)
                    xd = _vsel(cur, idxs)
                    cur = jnp.where(kv == kd, xd, cur)
                m = jnp.logical_and(kv >= w0s, kv < w1s)
                sidx = jnp.clip(kv - w0s, 0, wsize - 1)
                plsc.store_scatter(sb, [sidx], cur, mask=m)
                nlt = _popcount(kv < w1s)
                full = nlt == 16
                g3 = jnp.where(full, g2 + 16, g2)
                done2 = jnp.logical_or(jnp.logical_not(full), g3 >= TOT)
                cont2 = jnp.logical_and(
                    jnp.logical_not(done2), g3 - gb + 16 <= ESIZE)
                return (g3, done2, cont2)

            g4, done4, _ = lax.while_loop(cond, body, (g_, False, True))
            return (g4, done4)

        g_out, _ = lax.while_loop(refill_cond, refill_body, (g_init, False))
        return g_out

    def zero_buf(sb, nwords):
        def z(i, carry):
            sb[pl.ds(i * 16, 16)] = zeros16
            return carry
        lax.fori_loop(0, nwords // 16, z, 0, unroll=8)

    def do_window(win, g, b, wsize, first_round):
        sb, sem = sbufs[b], sems[b]
        w0 = tile_w0 + win * WSIZE
        if not first_round:
            pltpu.make_async_copy(
                sb.at[pl.ds(0, WSIZE)],
                y_hbm.at[pl.ds(pl.multiple_of(w0 - 2 * WSIZE, 16), WSIZE)],
                sem).wait()
        zero_buf(sb, WSIZE)
        g = consume_window(g, w0, w0 + wsize, sb, wsize)
        pltpu.async_copy(
            sb.at[pl.ds(0, wsize)],
            y_hbm.at[pl.ds(pl.multiple_of(w0, 16), wsize)], sem)
        return g

    def round0(g):
        g = do_window(0, g, 0, WSIZE, True)
        g = do_window(1, g, 1, WSIZE, True)
        return g

    def roundN(s, g):
        g = do_window(s * 2, g, 0, WSIZE, False)
        g = do_window(s * 2 + 1, g, 1, WSIZE, False)
        return g

    g = round0(g0)
    g = lax.fori_loop(1, 16, roundN, g)
    g = do_window(32, g, 0, WSIZE, False)

    w0t = tile_w0 + NWIN * WSIZE

    @pl.when(wid != NW - 1)
    def _():
        pltpu.make_async_copy(
            sb1.at[pl.ds(0, WSIZE)],
            y_hbm.at[pl.ds(pl.multiple_of(w0t - 2 * WSIZE, 16), WSIZE)],
            sem1).wait()
        zero_buf(sb1, TAIL)
        consume_window(g, w0t, w0t + TAIL, sb1, TAIL)
        pltpu.async_copy(
            sb1.at[pl.ds(0, TAIL)],
            y_hbm.at[pl.ds(pl.multiple_of(w0t, 16), TAIL)], sem1)
        pltpu.make_async_copy(
            sb1.at[pl.ds(0, TAIL)],
            y_hbm.at[pl.ds(pl.multiple_of(w0t, 16), TAIL)], sem1).wait()

    @pl.when(wid == NW - 1)
    def _():
        pltpu.make_async_copy(
            sb1.at[pl.ds(0, WSIZE)],
            y_hbm.at[pl.ds(pl.multiple_of(w0t - 2 * WSIZE, 16), WSIZE)],
            sem1).wait()
        zero_buf(sb1, TAIL31)
        consume_window(g, w0t, w0t + TAIL31, sb1, TAIL31)
        pltpu.async_copy(
            sb1.at[pl.ds(0, TAIL31)],
            y_hbm.at[pl.ds(pl.multiple_of(w0t, 16), TAIL31)], sem1)
        pltpu.make_async_copy(
            sb1.at[pl.ds(0, TAIL31)],
            y_hbm.at[pl.ds(pl.multiple_of(w0t, 16), TAIL31)], sem1).wait()

    pltpu.make_async_copy(
        sb0.at[pl.ds(0, WSIZE)],
        y_hbm.at[pl.ds(pl.multiple_of(tile_w0 + 32 * WSIZE, 16), WSIZE)],
        sem0).wait()


_scatter = pl.kernel(
    _body,
    out_type=jax.ShapeDtypeStruct((FLAT,), jnp.float32),
    mesh=plsc.VectorSubcoreMesh(core_axis_name="c", subcore_axis_name="s"),
    compiler_params=pltpu.CompilerParams(needs_layout_passes=False),
    scratch_types=(
        [pltpu.VMEM((WSIZE,), jnp.float32) for _ in range(2)]
        + [pltpu.VMEM((ESIZE,), jnp.int32), pltpu.VMEM((ESIZE,), jnp.float32)]
        + [pltpu.VMEM((16,), jnp.int32)]
        + [pltpu.SemaphoreType.DMA for _ in range(2)]
    ),
)


@jax.jit
def kernel(x, indices):
    idx32 = indices.astype(jnp.int32)
    keys = (idx32 * C + jnp.arange(C, dtype=jnp.int32)[None, :]).reshape(-1)
    skey, sval = lax.sort_key_val(keys, x.reshape(-1), is_stable=False)
    bounds = jnp.arange(NW, dtype=jnp.int32) * WPT
    starts = jnp.searchsorted(skey, bounds).astype(jnp.int32)
    starts_arr = jnp.broadcast_to(starts[:, None], (NW, 16))
    y = _scatter(skey, sval, starts_arr)
    return y.reshape(L, C)
